# Initial kernel scaffold; baseline (speedup 1.0000x reference)
import jax
import jax.numpy as jnp
from jax.experimental import pallas as pl


def kernel(x, edge_index, ratio, monomer_id, batch, task_id, g0_W1, g0_b1, g0_W2, g0_b2, g1_W1, g1_b1, g1_W2, g1_b2, g2_W1, g2_b1, g2_W2, g2_b2, fc_q_W, fc_q_b, fc_k_W, fc_k_b, fc_v_W, fc_v_b, o_W1, o_b1, o_W2, o_b2):
    raise NotImplementedError("write your pallas kernel here")


# speedup vs baseline: 3.7099x; 3.7099x over previous
import functools
import numpy as np
import jax
import jax.numpy as jnp
from jax import lax
from jax.experimental import pallas as pl
from jax.experimental.pallas import tpu as pltpu
from jax.experimental.pallas import tpu_sc as plsc

N = 10000
E = 320000
D = 128
NW = 32
CHUNK = 128
STEPS = -(-E // (NW * CHUNK))
EPT = STEPS * CHUNK
E_PAD = EPT * NW
M_ROWS = 2048
N_PAD = 10240
NROWS = N_PAD // 128
SP_ROWS = N_PAD + 16
ROWS_PER_TILE = N_PAD // 16


def _edge_agg_body(h_hbm, src_hbm, dst_hbm, zeros_hbm, out_hbm,
                   idx_s, idx_d, rows, agg_sh, sem):
    c = lax.axis_index("c")
    s = lax.axis_index("s")
    wid = s * 2 + c
    pltpu.sync_copy(zeros_hbm.at[pl.ds(s * ROWS_PER_TILE, ROWS_PER_TILE)],
                    agg_sh.at[pl.ds(s * ROWS_PER_TILE, ROWS_PER_TILE)])
    plsc.subcore_barrier()
    base = wid * EPT

    def step(j, carry):
        off = base + j * CHUNK
        pltpu.sync_copy(src_hbm.at[pl.ds(off, CHUNK)], idx_s)
        pltpu.sync_copy(dst_hbm.at[pl.ds(off, CHUNK)], idx_d)
        pltpu.async_copy(h_hbm.at[idx_s], rows, sem).wait()
        pltpu.sync_copy(rows, agg_sh.at[idx_d], add=True)
        return carry

    lax.fori_loop(0, STEPS, step, 0)
    plsc.subcore_barrier()
    pltpu.sync_copy(agg_sh.at[pl.ds(s * ROWS_PER_TILE, ROWS_PER_TILE)],
                    out_hbm.at[c, pl.ds(s * ROWS_PER_TILE, ROWS_PER_TILE)])


@functools.cache
def _edge_agg_kernel():
    return pl.kernel(
        _edge_agg_body,
        out_type=jax.ShapeDtypeStruct((2, N_PAD, D), jnp.float32),
        mesh=plsc.VectorSubcoreMesh(core_axis_name="c", subcore_axis_name="s"),
        scratch_types=[
            pltpu.VMEM((CHUNK,), jnp.int32),
            pltpu.VMEM((CHUNK,), jnp.int32),
            pltpu.VMEM((CHUNK, D), jnp.float32),
            pltpu.VMEM_SHARED((SP_ROWS, D), jnp.float32),
            pltpu.SemaphoreType.DMA,
        ],
    )


def _edge_agg(h, src_p, dst_p, zeros):
    return _edge_agg_kernel()(h, src_p, dst_p, zeros)


def _gin_mlp_body(h_ref, p0_ref, p1_ref, w1_ref, b1_ref, w2_ref, b2_ref, o_ref):
    z = h_ref[...] + p0_ref[...] + p1_ref[...]
    z = jnp.maximum(jnp.dot(z, w1_ref[...],
                            preferred_element_type=jnp.float32) + b1_ref[...], 0.0)
    z = jnp.dot(z, w2_ref[...], preferred_element_type=jnp.float32) + b2_ref[...]
    o_ref[...] = jnp.maximum(z, 0.0)


def _gin_mlp(h, p0, p1, w1, b1, w2, b2):
    blk = 1024
    grid = N_PAD // blk
    row_spec = pl.BlockSpec((blk, D), lambda i: (i, 0))
    full_spec = pl.BlockSpec((D, D), lambda i: (0, 0))
    bias_spec = pl.BlockSpec((1, D), lambda i: (0, 0))
    return pl.pallas_call(
        _gin_mlp_body,
        grid=(grid,),
        in_specs=[row_spec, row_spec, row_spec, full_spec, bias_spec,
                  full_spec, bias_spec],
        out_specs=row_spec,
        out_shape=jax.ShapeDtypeStruct((N_PAD, D), jnp.float32),
    )(h, p0, p1, w1, b1, w2, b2)


def _shift_down(a, s, fill):
    r, cdim = a.shape
    rs, cs = divmod(s, cdim)
    if rs > 0:
        a = jnp.concatenate(
            [jnp.full((rs, cdim), fill, a.dtype), a[:-rs]], axis=0)
    if cs > 0:
        prev = jnp.concatenate(
            [jnp.full((1, cdim), fill, a.dtype), a[:-1]], axis=0)
        a = jnp.concatenate([prev[:, cdim - cs:], a[:, :-cs]], axis=1)
    return a


def _shift_up(a, s, fill):
    r, cdim = a.shape
    rs, cs = divmod(s, cdim)
    if rs > 0:
        a = jnp.concatenate(
            [a[rs:], jnp.full((rs, cdim), fill, a.dtype)], axis=0)
    if cs > 0:
        nxt = jnp.concatenate(
            [a[1:], jnp.full((1, cdim), fill, a.dtype)], axis=0)
        a = jnp.concatenate([a[:, cs:], nxt[:, :cs]], axis=1)
    return a


def _rows_down(a, s, fill):
    return jnp.concatenate(
        [jnp.full((s, a.shape[1]), fill, a.dtype), a[:-s]], axis=0)


def _rows_up(a, s, fill):
    return jnp.concatenate(
        [a[s:], jnp.full((s, a.shape[1]), fill, a.dtype)], axis=0)


def _pool_attn_body(h_ref, mid_ref, ratio_ref, batch_ref,
                    qw_ref, qb_ref, kw_ref, kb_ref, vw_ref, vb_ref,
                    ow1_ref, ob1_ref, ow2_ref, ob2_ref, o_ref):
    mid = mid_ref[...]
    ratio = ratio_ref[...]
    batch = batch_ref[...]

    flat_idx = (lax.broadcasted_iota(jnp.int32, (NROWS, 128), 0) * 128
                + lax.broadcasted_iota(jnp.int32, (NROWS, 128), 1))
    not_first = (flat_idx > 0)

    prev_mid = _shift_down(mid, 1, 0)
    diff = jnp.where(not_first & (mid != prev_mid), 1, 0)

    mon = diff
    s = 1
    while s < N_PAD:
        mon = mon + _shift_down(mon, s, 0)
        s *= 2
    mon_f = mon.astype(jnp.float32)

    is_start = jnp.where(diff == 1, 1.0, 0.0) + jnp.where(flat_idx == 0, 1.0, 0.0)
    is_end = _shift_up(is_start, 1, 1.0)

    pmr = ratio
    s = 1
    while s < N_PAD:
        cand = _shift_down(pmr, s, jnp.inf)
        same = (mon_f == _shift_down(mon_f, s, -1.0))
        pmr = jnp.where(same, jnp.minimum(pmr, cand), pmr)
        s *= 2

    m_iota = lax.broadcasted_iota(jnp.int32, (M_ROWS, 128), 0).astype(jnp.float32)
    emb_sum = jnp.zeros((M_ROWS, D), jnp.float32)
    counts = jnp.zeros((M_ROWS, 1), jnp.float32)
    bsum = jnp.zeros((M_ROWS, 1), jnp.float32)
    fmin = jnp.zeros((M_ROWS, 1), jnp.float32)
    validity = jnp.zeros((M_ROWS, 1), jnp.float32)
    w_batch = batch * is_start
    w_fmin = pmr * is_end
    for r in range(NROWS):
        oh = jnp.where(m_iota == mon_f[r:r + 1, :], 1.0, 0.0)
        emb_sum = emb_sum + jnp.dot(oh, h_ref[pl.ds(r * 128, 128), :],
                                    preferred_element_type=jnp.float32)
        counts = counts + jnp.sum(oh, axis=1, keepdims=True)
        bsum = bsum + jnp.sum(oh * w_batch[r:r + 1, :], axis=1, keepdims=True)
        fmin = fmin + jnp.sum(oh * w_fmin[r:r + 1, :], axis=1, keepdims=True)
        validity = validity + jnp.sum(oh * is_start[r:r + 1, :], axis=1,
                                      keepdims=True)

    emb = emb_sum / jnp.maximum(counts, 1.0)
    bidx = bsum + 100.0 * (1.0 - validity)
    frac = fmin * validity

    q = (jnp.dot(emb, qw_ref[...], preferred_element_type=jnp.float32)
         + qb_ref[...]) * frac
    k_ = (jnp.dot(emb, kw_ref[...], preferred_element_type=jnp.float32)
          + kb_ref[...]) * frac
    v = jnp.dot(emb, vw_ref[...], preferred_element_type=jnp.float32) + vb_ref[...]

    g_iota = lax.broadcasted_iota(jnp.int32, (M_ROWS, 64), 1).astype(jnp.float32)
    ohT = jnp.where(g_iota == bidx, 1.0, 0.0)

    def group_sum(x):
        return lax.dot_general(ohT, x, (((0,), (0,)), ((), ())),
                               preferred_element_type=jnp.float32)

    k_sum = group_sum(k_)
    k_bc = jnp.dot(ohT, k_sum, preferred_element_type=jnp.float32)
    energy = q * k_bc * (1.0 / np.sqrt(128.0))

    fwd = energy
    bwd = energy
    s = 1
    while s < M_ROWS:
        same_d = (bidx == _rows_down(bidx, s, -1.0))
        cand = _rows_down(fwd, s, -jnp.inf)
        fwd = jnp.where(same_d, jnp.maximum(fwd, cand), fwd)
        same_u = (bidx == _rows_up(bidx, s, -1.0))
        cand = _rows_up(bwd, s, -jnp.inf)
        bwd = jnp.where(same_u, jnp.maximum(bwd, cand), bwd)
        s *= 2
    segmax = jnp.maximum(fwd, bwd)

    e = jnp.exp(energy - segmax)
    esum = group_sum(e)
    esum_bc = jnp.dot(ohT, esum, preferred_element_type=jnp.float32)
    att = e / (esum_bc + 1e-16)
    poly = group_sum(v * att)

    y = jnp.maximum(jnp.dot(poly, ow1_ref[...],
                            preferred_element_type=jnp.float32) + ob1_ref[...], 0.0)
    o_ref[...] = jnp.dot(y, ow2_ref[...],
                         preferred_element_type=jnp.float32) + ob2_ref[...]


def _pool_attn(h_pad, mid2d, ratio2d, batch2d, qw, qb, kw, kb, vw, vb,
               ow1, ob1, ow2p, ob2p):
    return pl.pallas_call(
        _pool_attn_body,
        out_shape=jax.ShapeDtypeStruct((64, 128), jnp.float32),
    )(h_pad, mid2d, ratio2d, batch2d, qw, qb, kw, kb, vw, vb, ow1, ob1,
      ow2p, ob2p)


def kernel(x, edge_index, ratio, monomer_id, batch, task_id,
           g0_W1, g0_b1, g0_W2, g0_b2,
           g1_W1, g1_b1, g1_W2, g1_b2,
           g2_W1, g2_b1, g2_W2, g2_b2,
           fc_q_W, fc_q_b, fc_k_W, fc_k_b, fc_v_W, fc_v_b,
           o_W1, o_b1, o_W2, o_b2):
    src = edge_index[0]
    dst = edge_index[1]
    pad = E_PAD - E
    src_p = jnp.concatenate([src, jnp.zeros((pad,), jnp.int32)])
    dst_p = jnp.concatenate([dst, jnp.full((pad,), N_PAD, jnp.int32)])
    zeros = jnp.zeros((N_PAD, D), jnp.float32)
    npad = N_PAD - N

    gin = [(g0_W1, g0_b1, g0_W2, g0_b2),
           (g1_W1, g1_b1, g1_W2, g1_b2),
           (g2_W1, g2_b1, g2_W2, g2_b2)]
    h = jnp.concatenate([x, jnp.zeros((npad, D), jnp.float32)], axis=0)
    for (w1, b1, w2, b2) in gin:
        parts = _edge_agg(h, src_p, dst_p, zeros)
        h = _gin_mlp(h, parts[0], parts[1], w1, b1.reshape(1, D),
                     w2, b2.reshape(1, D))

    mid2d = jnp.concatenate(
        [monomer_id, jnp.full((npad,), 2_000_000, jnp.int32)]).reshape(NROWS, 128)
    ratio2d = jnp.concatenate(
        [ratio, jnp.ones((npad,), jnp.float32)]).reshape(NROWS, 128)
    batch2d = jnp.concatenate(
        [batch.astype(jnp.float32),
         jnp.full((npad,), 100.0, jnp.float32)]).reshape(NROWS, 128)
    ow2p = jnp.zeros((D, 128), jnp.float32).at[:, 0].set(o_W2[:, 0])
    ob2p = jnp.zeros((1, 128), jnp.float32).at[0, 0].set(o_b2[0])

    out = _pool_attn(h, mid2d, ratio2d, batch2d,
                     fc_q_W, fc_q_b.reshape(1, D), fc_k_W, fc_k_b.reshape(1, D),
                     fc_v_W, fc_v_b.reshape(1, D), o_W1, o_b1.reshape(1, D),
                     ow2p, ob2p)
    return out[:, 0]
